# baseline (speedup 1.0000x reference)
import functools

import jax
import jax.numpy as jnp
from jax import lax
from jax.experimental import pallas as pl
from jax.experimental.pallas import tpu as pltpu
from jax.experimental.pallas import tpu_sc as plsc

_NC = 2
_NS = 16
_U = 5
_CW = 16


def _sc_agg(tab, src1, dst1, with_cnt, c):
    n, d = tab.shape
    dt = tab.dtype
    e = src1.shape[0]
    nw = _NC * _NS
    nchunks = e // c // nw
    ngrp = nchunks // _U
    assert nchunks * c * nw == e and ngrp * _U == nchunks
    full_dst = not with_cnt
    stripe = 640
    last_stripe = n - stripe * (_NS - 1)
    assert last_stripe > 0 and last_stripe % 8 == 0
    f32 = jnp.float32

    out_type = [jax.ShapeDtypeStruct((n, d), dt) for _ in range(_NC)]
    scratch = [
        pltpu.VMEM((nchunks, c), jnp.int32),
        pltpu.VMEM((nchunks if full_dst else _U, c), jnp.int32),
        pltpu.VMEM((c, d), dt),
        pltpu.VMEM((c, d), dt),
        pltpu.VMEM_SHARED((n, d), dt),
        pltpu.SemaphoreType.DMA,
        pltpu.SemaphoreType.DMA,
        pltpu.SemaphoreType.DMA,
        pltpu.SemaphoreType.DMA,
    ]
    if with_cnt:
        out_type += [jax.ShapeDtypeStruct((n, _CW), f32) for _ in range(_NC)]
        scratch += [
            pltpu.VMEM((c, _CW), f32),
            pltpu.VMEM_SHARED((n, _CW), f32),
        ]

    mesh = plsc.VectorSubcoreMesh(core_axis_name="c", subcore_axis_name="s")
    cparams = pltpu.CompilerParams(use_tc_tiling_on_sc=False)

    @functools.partial(pl.kernel, mesh=mesh, out_type=out_type,
                       scratch_types=scratch, compiler_params=cparams)
    def k(tab_h, src_h, dst_h, zd_h, zc_h, *rest):
        if with_cnt:
            (o0, o1, oc0, oc1, srcb, dstb, rows0, rows1, acc,
             gsem0, gsem1, ssem0, ssem1, ones_v, accc) = rest
        else:
            (o0, o1, srcb, dstb, rows0, rows1, acc,
             gsem0, gsem1, ssem0, ssem1) = rest
            ones_v = accc = None
        rows = (rows0, rows1)
        gsems = (gsem0, gsem1)
        ssems = (ssem0, ssem1)
        cid = lax.axis_index("c")
        sid = lax.axis_index("s")
        wid = cid * _NS + sid
        r0 = sid * stripe

        def _each_stripe(fn):
            @pl.when(sid < _NS - 1)
            def _():
                fn(pl.ds(r0, stripe))

            @pl.when(sid == _NS - 1)
            def _():
                fn(pl.ds((_NS - 1) * stripe, last_stripe))

        def _init(rsl):
            pltpu.sync_copy(zd_h.at[rsl], acc.at[rsl])
            if with_cnt:
                pltpu.sync_copy(zc_h.at[rsl], accc.at[rsl])

        _each_stripe(_init)
        base = wid * nchunks
        pltpu.sync_copy(src_h.at[pl.ds(base, nchunks)], srcb)
        if full_dst:
            pltpu.sync_copy(dst_h.at[pl.ds(base, nchunks)], dstb)
        if with_cnt:
            for i in range(c):
                ones_v[i] = jnp.ones((_CW,), f32)
        plsc.subcore_barrier()

        def _drain(b):
            pltpu.make_async_copy(zd_h.at[pl.ds(0, c)], rows[b],
                                  ssems[b]).wait()
            if with_cnt:
                pltpu.make_async_copy(zc_h.at[pl.ds(0, c)], ones_v,
                                      ssems[b]).wait()

        def _scatter(u, dsl):
            b = u % 2
            pltpu.async_copy(rows[b], acc.at[dsl], ssems[b], add=True)
            if with_cnt:
                pltpu.async_copy(ones_v, accc.at[dsl], ssems[b], add=True)

        def do_group(g, first):
            j0 = g * _U
            if not first:
                _drain(0)
                _drain(1)
            if not full_dst:
                pltpu.sync_copy(dst_h.at[pl.ds(base + j0, _U)], dstb)
            handles = [None] * _U
            for u in range(_U):
                b = u % 2
                if u >= 2:
                    _drain(b)
                handles[u] = pltpu.async_copy(
                    tab_h.at[srcb.at[j0 + u]], rows[b], gsems[b])
                if u > 0:
                    handles[u - 1].wait()
                    _scatter(u - 1,
                             dstb.at[(j0 + u - 1) if full_dst else (u - 1)])
            handles[_U - 1].wait()
            _scatter(_U - 1, dstb.at[(j0 + _U - 1) if full_dst else (_U - 1)])

        do_group(0, True)

        def grp(g, carry):
            do_group(g, False)
            return carry

        lax.fori_loop(1, ngrp, grp, 0)
        _drain(0)
        _drain(1)
        plsc.subcore_barrier()

        def _writeout(rsl):
            @pl.when(cid == 0)
            def _():
                pltpu.sync_copy(acc.at[rsl], o0.at[rsl])
                if with_cnt:
                    pltpu.sync_copy(accc.at[rsl], oc0.at[rsl])

            @pl.when(cid == 1)
            def _():
                pltpu.sync_copy(acc.at[rsl], o1.at[rsl])
                if with_cnt:
                    pltpu.sync_copy(accc.at[rsl], oc1.at[rsl])

        _each_stripe(_writeout)

    zd = jnp.zeros((n, d), dt)
    zc = jnp.zeros((n, _CW), f32)
    return k(tab, src1.reshape(-1, c), dst1.reshape(-1, c), zd, zc)


def _xr_body(xv, wr1, bl1, o):
    o[...] = (jnp.dot(xv[...], wr1[...], preferred_element_type=jnp.float32)
              + bl1[...])


def _dense1_body(a0, a1, c0, c1, xr, wl1, g1r, be1r, wl2, oh2, oh, ocnt):
    cnt = jnp.maximum((c0[...] + c1[...])[:, 0:1], 1.0)
    mean1 = (a0[...].astype(jnp.float32) + a1[...].astype(jnp.float32)) / cnt
    h = (jnp.dot(mean1, wl1[...], preferred_element_type=jnp.float32)
         + xr[...])
    mu = jnp.mean(h, axis=0, keepdims=True)
    var = jnp.mean((h - mu) ** 2, axis=0, keepdims=True)
    h = (h - mu) / jnp.sqrt(var + 1e-5) * g1r[...] + be1r[...]
    h = jnp.maximum(h, 0.0)
    oh2[...] = jnp.dot(h, wl2[...],
                       preferred_element_type=jnp.float32).astype(jnp.bfloat16)
    oh[...] = h
    ocnt[...] = cnt


def _hr_body(hv, wr2, o):
    o[...] = jnp.dot(hv[...], wr2[...], preferred_element_type=jnp.float32)


def _dense2_body(q0, q1, cnt_r, hr, bl2, g2r, be2r, wc, bc, out):
    h = ((q0[...].astype(jnp.float32) + q1[...].astype(jnp.float32))
         / cnt_r[...] + bl2[...] + hr[...])
    mu = jnp.mean(h, axis=0, keepdims=True)
    var = jnp.mean((h - mu) ** 2, axis=0, keepdims=True)
    h = (h - mu) / jnp.sqrt(var + 1e-5) * g2r[...] + be2r[...]
    h = jnp.maximum(h, 0.0)
    out[...] = jnp.dot(h, wc[...], preferred_element_type=jnp.float32) + bc[...]


def kernel(x, edge_index, W_l1, b_l1, W_r1, g1, be1, W_l2, b_l2, W_r2,
           g2, be2, Wc, bc):
    n, d = x.shape
    e = edge_index.shape[1]
    h2 = W_l2.shape[0]
    f32 = jnp.float32

    src1 = edge_index[0]
    dst1 = edge_index[1]

    a0, a1, c0, c1 = _sc_agg(x.astype(jnp.bfloat16), src1, dst1,
                             with_cnt=True, c=80)

    xr = pl.pallas_call(
        _xr_body, out_shape=jax.ShapeDtypeStruct((n, d), f32),
    )(x, W_r1.T, b_l1.reshape(1, -1))

    dense1 = pl.pallas_call(
        _dense1_body,
        out_shape=[
            jax.ShapeDtypeStruct((n, h2), jnp.bfloat16),
            jax.ShapeDtypeStruct((n, d), f32),
            jax.ShapeDtypeStruct((n, 1), f32),
        ],
    )
    h2pre, h1, cnt = dense1(
        a0, a1, c0, c1, xr, W_l1.T,
        g1.reshape(1, -1), be1.reshape(1, -1), W_l2.T)

    q0, q1 = _sc_agg(h2pre, src1, dst1, with_cnt=False, c=80)

    hr = pl.pallas_call(
        _hr_body, out_shape=jax.ShapeDtypeStruct((n, h2), f32),
    )(h1, W_r2.T)

    wc_pad = jnp.zeros((h2, 8), f32).at[:, :2].set(Wc.T)
    bc_pad = jnp.zeros((1, 8), f32).at[0, :2].set(bc)
    dense2 = pl.pallas_call(
        _dense2_body,
        out_shape=jax.ShapeDtypeStruct((n, 8), f32),
    )
    logits8 = dense2(q0, q1, cnt, hr, b_l2.reshape(1, -1),
                     g2.reshape(1, -1), be2.reshape(1, -1), wc_pad, bc_pad)
    return logits8[:, :2]


# speedup vs baseline: 1.0018x; 1.0018x over previous
import functools

import jax
import jax.numpy as jnp
from jax import lax
from jax.experimental import pallas as pl
from jax.experimental.pallas import tpu as pltpu
from jax.experimental.pallas import tpu_sc as plsc

_NC = 2
_NS = 16
_U = 5
_CW = 16


def _sc_agg(tab, src1, dst1, with_cnt, c):
    n, d = tab.shape
    dt = tab.dtype
    e = src1.shape[0]
    nw = _NC * _NS
    nchunks = e // c // nw
    ngrp = nchunks // _U
    assert nchunks * c * nw == e and ngrp * _U == nchunks
    full_dst = not with_cnt
    stripe = 640
    last_stripe = n - stripe * (_NS - 1)
    assert last_stripe > 0 and last_stripe % 8 == 0
    f32 = jnp.float32

    out_type = [jax.ShapeDtypeStruct((n, d), dt) for _ in range(_NC)]
    scratch = [
        pltpu.VMEM((nchunks, c), jnp.int32),
        pltpu.VMEM((nchunks if full_dst else _U, c), jnp.int32),
        pltpu.VMEM((c, d), dt),
        pltpu.VMEM((c, d), dt),
        pltpu.VMEM_SHARED((n, d), dt),
        pltpu.SemaphoreType.DMA,
        pltpu.SemaphoreType.DMA,
        pltpu.SemaphoreType.DMA,
        pltpu.SemaphoreType.DMA,
    ]
    if with_cnt:
        out_type += [jax.ShapeDtypeStruct((n, _CW), f32) for _ in range(_NC)]
        scratch += [
            pltpu.VMEM((c, _CW), f32),
            pltpu.VMEM_SHARED((n, _CW), f32),
        ]

    mesh = plsc.VectorSubcoreMesh(core_axis_name="c", subcore_axis_name="s")
    cparams = pltpu.CompilerParams(use_tc_tiling_on_sc=False)

    @functools.partial(pl.kernel, mesh=mesh, out_type=out_type,
                       scratch_types=scratch, compiler_params=cparams)
    def k(tab_h, src_h, dst_h, zd_h, zc_h, *rest):
        if with_cnt:
            (o0, o1, oc0, oc1, srcb, dstb, rows0, rows1, acc,
             gsem0, gsem1, ssem0, ssem1, ones_v, accc) = rest
        else:
            (o0, o1, srcb, dstb, rows0, rows1, acc,
             gsem0, gsem1, ssem0, ssem1) = rest
            ones_v = accc = None
        rows = (rows0, rows1)
        gsems = (gsem0, gsem1)
        ssems = (ssem0, ssem1)
        cid = lax.axis_index("c")
        sid = lax.axis_index("s")
        wid = cid * _NS + sid
        r0 = sid * stripe

        def _each_stripe(fn):
            @pl.when(sid < _NS - 1)
            def _():
                fn(pl.ds(r0, stripe))

            @pl.when(sid == _NS - 1)
            def _():
                fn(pl.ds((_NS - 1) * stripe, last_stripe))

        def _init(rsl):
            pltpu.sync_copy(zd_h.at[rsl], acc.at[rsl])
            if with_cnt:
                pltpu.sync_copy(zc_h.at[rsl], accc.at[rsl])

        _each_stripe(_init)
        base = wid * nchunks
        pltpu.sync_copy(src_h.at[pl.ds(base, nchunks)], srcb)
        if full_dst:
            pltpu.sync_copy(dst_h.at[pl.ds(base, nchunks)], dstb)
        if with_cnt:
            for i in range(c):
                ones_v[i] = jnp.ones((_CW,), f32)
        plsc.subcore_barrier()

        def _drain(b):
            pltpu.make_async_copy(zd_h.at[pl.ds(0, c)], rows[b],
                                  ssems[b]).wait()
            if with_cnt:
                pltpu.make_async_copy(zc_h.at[pl.ds(0, c)], ones_v,
                                      ssems[b]).wait()

        def _scatter(u, dsl):
            b = u % 2
            pltpu.async_copy(rows[b], acc.at[dsl], ssems[b], add=True)
            if with_cnt:
                pltpu.async_copy(ones_v, accc.at[dsl], ssems[b], add=True)

        def do_group(g, first):
            j0 = g * _U
            if not first:
                _drain(0)
                _drain(1)
            if not full_dst:
                pltpu.sync_copy(dst_h.at[pl.ds(base + j0, _U)], dstb)
            handles = [None] * _U
            for u in range(_U):
                b = u % 2
                if u >= 2:
                    _drain(b)
                handles[u] = pltpu.async_copy(
                    tab_h.at[srcb.at[j0 + u]], rows[b], gsems[b])
                if u > 0:
                    handles[u - 1].wait()
                    _scatter(u - 1,
                             dstb.at[(j0 + u - 1) if full_dst else (u - 1)])
            handles[_U - 1].wait()
            _scatter(_U - 1, dstb.at[(j0 + _U - 1) if full_dst else (_U - 1)])

        do_group(0, True)

        def grp(g, carry):
            do_group(g, False)
            return carry

        lax.fori_loop(1, ngrp, grp, 0)
        _drain(0)
        _drain(1)
        plsc.subcore_barrier()

        def _writeout(rsl):
            @pl.when(cid == 0)
            def _():
                pltpu.sync_copy(acc.at[rsl], o0.at[rsl])
                if with_cnt:
                    pltpu.sync_copy(accc.at[rsl], oc0.at[rsl])

            @pl.when(cid == 1)
            def _():
                pltpu.sync_copy(acc.at[rsl], o1.at[rsl])
                if with_cnt:
                    pltpu.sync_copy(accc.at[rsl], oc1.at[rsl])

        _each_stripe(_writeout)

    zd = jnp.zeros((n, d), dt)
    zc = jnp.zeros((n, _CW), f32)
    return k(tab, src1.reshape(-1, c), dst1.reshape(-1, c), zd, zc)


def _dense1_body(a0, a1, c0, c1, xr, wl1, bl1, wr1, g1r, be1r, wl2, wr2,
                 oh2, ohr, ocnt):
    cnt = jnp.maximum((c0[...] + c1[...])[:, 0:1], 1.0)
    mean1 = (a0[...].astype(jnp.float32) + a1[...].astype(jnp.float32)) / cnt
    h = (jnp.dot(mean1, wl1[...], preferred_element_type=jnp.float32)
         + bl1[...]
         + jnp.dot(xr[...], wr1[...], preferred_element_type=jnp.float32))
    mu = jnp.mean(h, axis=0, keepdims=True)
    var = jnp.mean((h - mu) ** 2, axis=0, keepdims=True)
    h = (h - mu) / jnp.sqrt(var + 1e-5) * g1r[...] + be1r[...]
    h = jnp.maximum(h, 0.0)
    oh2[...] = jnp.dot(h, wl2[...],
                       preferred_element_type=jnp.float32).astype(jnp.bfloat16)
    ohr[...] = jnp.dot(h, wr2[...], preferred_element_type=jnp.float32)
    ocnt[...] = cnt


def _dense2_body(q0, q1, cnt_r, hr, bl2, g2r, be2r, wc, bc, out):
    h = ((q0[...].astype(jnp.float32) + q1[...].astype(jnp.float32))
         / cnt_r[...] + bl2[...] + hr[...])
    mu = jnp.mean(h, axis=0, keepdims=True)
    var = jnp.mean((h - mu) ** 2, axis=0, keepdims=True)
    h = (h - mu) / jnp.sqrt(var + 1e-5) * g2r[...] + be2r[...]
    h = jnp.maximum(h, 0.0)
    out[...] = jnp.dot(h, wc[...], preferred_element_type=jnp.float32) + bc[...]


def kernel(x, edge_index, W_l1, b_l1, W_r1, g1, be1, W_l2, b_l2, W_r2,
           g2, be2, Wc, bc):
    n, d = x.shape
    e = edge_index.shape[1]
    h2 = W_l2.shape[0]
    f32 = jnp.float32

    src1 = edge_index[0]
    dst1 = edge_index[1]

    a0, a1, c0, c1 = _sc_agg(x.astype(jnp.bfloat16), src1, dst1,
                             with_cnt=True, c=80)

    dense1 = pl.pallas_call(
        _dense1_body,
        out_shape=[
            jax.ShapeDtypeStruct((n, h2), jnp.bfloat16),
            jax.ShapeDtypeStruct((n, h2), f32),
            jax.ShapeDtypeStruct((n, 1), f32),
        ],
    )
    h2pre, hr, cnt = dense1(
        a0, a1, c0, c1, x, W_l1.T, b_l1.reshape(1, -1), W_r1.T,
        g1.reshape(1, -1), be1.reshape(1, -1), W_l2.T, W_r2.T)

    q0, q1 = _sc_agg(h2pre, src1, dst1, with_cnt=False, c=80)

    wc_pad = jnp.zeros((h2, 8), f32).at[:, :2].set(Wc.T)
    bc_pad = jnp.zeros((1, 8), f32).at[0, :2].set(bc)
    dense2 = pl.pallas_call(
        _dense2_body,
        out_shape=jax.ShapeDtypeStruct((n, 8), f32),
    )
    logits8 = dense2(q0, q1, cnt, hr, b_l2.reshape(1, -1),
                     g2.reshape(1, -1), be2.reshape(1, -1), wc_pad, bc_pad)
    return logits8[:, :2]
